# Initial kernel scaffold; baseline (speedup 1.0000x reference)
import jax
import jax.numpy as jnp
from jax.experimental import pallas as pl


def kernel(x, edge_index, W_l0, b_l0, W_r0, gamma0, beta0, W_l1, b_l1, W_r1, gamma1, beta1):
    raise NotImplementedError("write your pallas kernel here")


# speedup vs baseline: 1.2208x; 1.2208x over previous
import functools
import jax
import jax.numpy as jnp
from jax.experimental import pallas as pl
from jax.experimental.pallas import tpu as pltpu

N = 10000
E = 320000
D = 128


def _dense_body(M_ref, d2_ref, msk_ref, x_ref, Wa_ref, Wb_ref, wd_ref, bl_ref,
                Wr_ref, g_ref, b_ref, res_ref, o_ref, *, add_residual):
    x = x_ref[...]
    M = M_ref[...]
    dmax = jnp.sqrt(d2_ref[...])
    msk = msk_ref[...]
    agg_part = M @ Wa_ref[...] - x @ Wb_ref[...] + dmax * wd_ref[...]
    pre = msk * agg_part + bl_ref[...] + x @ Wr_ref[...]
    mean = jnp.mean(pre, axis=0, keepdims=True)
    var = jnp.mean((pre - mean) ** 2, axis=0, keepdims=True)
    h = (pre - mean) / jnp.sqrt(var + 1e-5) * g_ref[...] + b_ref[...]
    h = jnp.where(h >= 0, h, 0.01 * h)
    if add_residual:
        h = h + res_ref[...]
    o_ref[...] = h


def _dense_layer(M, d2, msk, x, W_l, b_l, W_r, gamma, beta, res, add_residual):
    Wa = W_l[:D] + W_l[D:2 * D]
    Wb = W_l[D:2 * D]
    wd = W_l[2 * D:2 * D + 1]
    return pl.pallas_call(
        functools.partial(_dense_body, add_residual=add_residual),
        out_shape=jax.ShapeDtypeStruct((N, D), jnp.float32),
    )(M, d2.reshape(N, 1), msk.reshape(N, 1), x, Wa, Wb, wd,
      b_l.reshape(1, D), W_r, gamma.reshape(1, D), beta.reshape(1, D), res)


def _sparse_agg(x, src, dst):
    xj = jnp.take(x, src, axis=0)
    xi = jnp.take(x, dst, axis=0)
    diff = xj - xi
    d2e = jnp.sum(diff * diff, axis=-1)
    M = jax.ops.segment_max(xj, dst, num_segments=N)
    d2 = jax.ops.segment_max(d2e, dst, num_segments=N)
    msk = jax.ops.segment_max(jnp.ones((E,), jnp.float32), dst, num_segments=N)
    M = jnp.where(jnp.isfinite(M), M, 0.0)
    d2 = jnp.where(jnp.isfinite(d2), d2, 0.0)
    msk = jnp.where(jnp.isfinite(msk), msk, 0.0)
    return M, d2, msk


@jax.jit
def kernel(x, edge_index, W_l0, b_l0, W_r0, gamma0, beta0, W_l1, b_l1, W_r1,
           gamma1, beta1):
    src = edge_index[0].astype(jnp.int32)
    dst = edge_index[1].astype(jnp.int32)
    M, d2, msk = _sparse_agg(x, src, dst)
    h = _dense_layer(M, d2, msk, x, W_l0, b_l0, W_r0, gamma0, beta0, x, False)
    M, d2, msk = _sparse_agg(h, src, dst)
    out = _dense_layer(M, d2, msk, h, W_l1, b_l1, W_r1, gamma1, beta1, x, True)
    return out
